# baseline (speedup 1.0000x reference)
import functools

import jax
import jax.numpy as jnp
from jax import lax
from jax.experimental import pallas as pl
from jax.experimental.pallas import tpu as pltpu
from jax.experimental.pallas import tpu_sc as plsc

NC, NS, L = 2, 16, 16
NBATCH = 4
N_REAL = 2500
N_PADB = 2560
NP = NBATCH * N_PADB
H = 128
ER = 2560
EPT = ER // NS
EPW = ER // (NS * NC)
RPT = NP // NS
CH = 16
EC = 16
GR = 64
EQT = 2 * EPT
EQW = 2 * EPW
CHD = 32
NBUF = 3


def _fast_rsqrt(d):
    i = lax.bitcast_convert_type(d, jnp.int32)
    y = lax.bitcast_convert_type(jnp.int32(0x5F3759DF) - (i >> 1), jnp.float32)
    for _ in range(3):
        y = y * (1.5 - 0.5 * d * y * y)
    return y


def _sc_body(srcp, dstp, srcq, dstq, hpad, g_out, agg_out,
             sbuf, dbuf, sbuf2, dbuf2, hb0, hb1,
             r0, r1, r2, ones_v, ns_v, nd_v,
             dout_sh, din_sh, acc_sh,
             sem_c, g0, g1, g2, s0, s1, s2):
    c = lax.axis_index("c")
    s = lax.axis_index("s")

    def _zrow(j, _):
        for k in range(8):
            hb0[j, pl.ds(k * L, L)] = jnp.zeros((L,), jnp.float32)
        return 0
    lax.fori_loop(0, CH, _zrow, 0)
    for k in range(8):
        ones_v[pl.ds(k * L, L)] = jnp.ones((L,), jnp.float32)

    zdescs = []
    for k in range(RPT // H):
        zdescs.append(pltpu.async_copy(
            hb0.at[0], dout_sh.at[pl.ds(s * RPT + k * H, H)], sem_c))
        zdescs.append(pltpu.async_copy(
            hb0.at[0], din_sh.at[pl.ds(s * RPT + k * H, H)], sem_c))
    for t in range(RPT // CH):
        zdescs.append(pltpu.async_copy(
            hb0, acc_sh.at[pl.ds(s * RPT + t * CH, CH)], sem_c))
    for d in zdescs:
        d.wait()
    plsc.subcore_barrier()

    for t in range(EPT // EC):
        pltpu.sync_copy(srcp.at[pl.ds(s * EPT + t * EC, EC)], sbuf)
        pltpu.sync_copy(dstp.at[pl.ds(s * EPT + t * EC, EC)], dbuf)
        descs = []
        for i in range(EC):
            descs.append(pltpu.async_copy(
                ones_v, dout_sh.at[sbuf.at[i]], sem_c, add=True))
            descs.append(pltpu.async_copy(
                ones_v, din_sh.at[dbuf.at[i]], sem_c, add=True))
        for d in descs:
            d.wait()
    plsc.subcore_barrier()

    pltpu.sync_copy(dout_sh.at[pl.ds(s * RPT, RPT)], ns_v)
    pltpu.sync_copy(din_sh.at[pl.ds(s * RPT, RPT)], nd_v)

    def _norm(i, _):
        sl = pl.ds(i * L, L)
        ns_v[sl] = _fast_rsqrt(jnp.maximum(ns_v[sl], 1.0))
        nd_v[sl] = _fast_rsqrt(jnp.maximum(nd_v[sl], 1.0))
        return 0
    lax.fori_loop(0, RPT // L, _norm, 0)

    hb = (hb0, hb1)
    isems = (g0, g1)
    osems = (s0, s1)
    NCH = RPT // CH
    din = [None, None]
    dout = [None, None]
    din[0] = pltpu.async_copy(hpad.at[pl.ds(s * RPT, CH)], hb0, g0)
    for t in range(NCH):
        cur = t % 2
        nxt = 1 - cur
        if t + 1 < NCH:
            if dout[nxt] is not None:
                dout[nxt].wait()
            din[nxt] = pltpu.async_copy(
                hpad.at[pl.ds(s * RPT + (t + 1) * CH, CH)], hb[nxt],
                isems[nxt])
        din[cur].wait()

        def _scale(j, _, t=t, cur=cur):
            idx = jnp.zeros((L,), jnp.int32) + (t * CH + j)
            sv = plsc.load_gather(ns_v, [idx])
            for k in range(8):
                sl = pl.ds(k * L, L)
                hb[cur][j, sl] = hb[cur][j, sl] * sv
            return 0
        lax.fori_loop(0, CH, _scale, 0)
        dout[cur] = pltpu.async_copy(
            hb[cur], g_out.at[pl.ds(s * RPT + t * CH, CH)], osems[cur])
    for d in dout:
        if d is not None:
            d.wait()
    plsc.subcore_barrier()

    bufs = (r0, r1, r2)
    gsems = (g0, g1, g2)
    ssems = (s0, s1, s2)
    for t in range(EQW // CHD):
        base = s * EQT + c * EQW + t * CHD
        pltpu.sync_copy(srcq.at[pl.ds(base, CHD)], sbuf2)
        pltpu.sync_copy(dstq.at[pl.ds(base, CHD)], dbuf2)
        dg = [None] * NBUF
        sc = [None] * NBUF
        for i in range(NBUF - 1):
            dg[i] = pltpu.async_copy(g_out.at[sbuf2.at[i]], bufs[i],
                                     gsems[i])
        for i in range(CHD):
            b = i % NBUF
            dg[b].wait()
            sc[b] = pltpu.async_copy(
                bufs[b], acc_sh.at[dbuf2.at[i]], ssems[b], add=True)
            j = i + NBUF - 1
            if j < CHD:
                bj = j % NBUF
                if sc[bj] is not None:
                    sc[bj].wait()
                dg[bj] = pltpu.async_copy(g_out.at[sbuf2.at[j]], bufs[bj],
                                          gsems[bj])
        for d in sc:
            if d is not None:
                d.wait()
    plsc.subcore_barrier()

    din = [None, None]
    dout = [None, None]
    din[0] = pltpu.async_copy(acc_sh.at[pl.ds(s * RPT, CH)], hb0, g0)
    for t in range(NCH):
        cur = t % 2
        nxt = 1 - cur
        if t + 1 < NCH:
            if dout[nxt] is not None:
                dout[nxt].wait()
            din[nxt] = pltpu.async_copy(
                acc_sh.at[pl.ds(s * RPT + (t + 1) * CH, CH)], hb[nxt],
                isems[nxt])
        din[cur].wait()

        def _scale_out(j, _, t=t, cur=cur):
            idx = jnp.zeros((L,), jnp.int32) + (t * CH + j)
            sv = plsc.load_gather(nd_v, [idx])
            for k in range(8):
                sl = pl.ds(k * L, L)
                hb[cur][j, sl] = hb[cur][j, sl] * sv
            return 0
        lax.fori_loop(0, CH, _scale_out, 0)
        dout[cur] = pltpu.async_copy(
            hb[cur], agg_out.at[pl.ds(c * NP + s * RPT + t * CH, CH)],
            osems[cur])
    for d in dout:
        if d is not None:
            d.wait()


def _sc_aggregate(srcp, dstp, srcq, dstq, hpad):
    mesh = plsc.VectorSubcoreMesh(core_axis_name="c", subcore_axis_name="s")
    f = pl.kernel(
        _sc_body,
        out_type=(
            jax.ShapeDtypeStruct((NP, H), jnp.float32),
            jax.ShapeDtypeStruct((NC * NP, H), jnp.float32),
        ),
        mesh=mesh,
        compiler_params=pltpu.CompilerParams(needs_layout_passes=False),
        scratch_types=[
            pltpu.VMEM((EC, H), jnp.int32),
            pltpu.VMEM((EC, H), jnp.int32),
            pltpu.VMEM((CHD, GR), jnp.int32),
            pltpu.VMEM((CHD, GR), jnp.int32),
            pltpu.VMEM((CH, H), jnp.float32),
            pltpu.VMEM((CH, H), jnp.float32),
            pltpu.VMEM((GR, H), jnp.float32),
            pltpu.VMEM((GR, H), jnp.float32),
            pltpu.VMEM((GR, H), jnp.float32),
            pltpu.VMEM((H,), jnp.float32),
            pltpu.VMEM((RPT,), jnp.float32),
            pltpu.VMEM((RPT,), jnp.float32),
            pltpu.VMEM_SHARED((NP,), jnp.float32),
            pltpu.VMEM_SHARED((NP,), jnp.float32),
            pltpu.VMEM_SHARED((NP, H), jnp.float32),
        ] + [pltpu.SemaphoreType.DMA] * 7,
    )
    return f(srcp, dstp, srcq, dstq, hpad)


BR = 512


def _tc_body(agg_ref, h_ref, x_ref, wr_ref, br_ref, wz_ref, bz_ref,
             wh_ref, bh_ref, gw_ref, gb_ref, out_ref):
    a = agg_ref[0] + agg_ref[1]
    hc = jnp.dot(a, gw_ref[...], preferred_element_type=jnp.float32)
    hc = hc + gb_ref[...][None, :]

    x = x_ref[...]
    dn = (((1,), (1,)), ((), ()))
    xr = lax.dot_general(x, wr_ref[...], dn,
                         preferred_element_type=jnp.float32) + br_ref[...][None, :]
    xz = lax.dot_general(x, wz_ref[...], dn,
                         preferred_element_type=jnp.float32) + bz_ref[...][None, :]
    xh = lax.dot_general(x, wh_ref[...], dn,
                         preferred_element_type=jnp.float32) + bh_ref[...][None, :]

    bid = pl.program_id(0)
    onehot = (lax.broadcasted_iota(jnp.int32, (NBATCH, 1), 0) == bid
              ).astype(jnp.float32)
    xr_b = jnp.sum(xr * onehot, axis=0, keepdims=True)
    xz_b = jnp.sum(xz * onehot, axis=0, keepdims=True)
    xh_b = jnp.sum(xh * onehot, axis=0, keepdims=True)

    r_t = jax.nn.sigmoid(xr_b + hc)
    z_t = jax.nn.sigmoid(xz_b + hc)
    h_tilde = jnp.tanh(xh_b + r_t * hc)
    out_ref[0] = (1.0 - z_t) * h_ref[0] + z_t * h_tilde


def _tc_gru(aggp, h_prev, x, w_r, b_r, w_z, b_z, w_h, b_h, gcn_w, gcn_b):
    KB = N_PADB // BR
    full2 = lambda shape: pl.BlockSpec(
        shape, lambda b, k: tuple(0 for _ in shape))
    return pl.pallas_call(
        _tc_body,
        grid=(NBATCH, KB),
        in_specs=[
            pl.BlockSpec((NC, BR, H), lambda b, k: (0, b * KB + k, 0)),
            pl.BlockSpec((1, BR, H), lambda b, k: (b, k, 0)),
            full2((NBATCH, H)),
            full2((H, H)), full2((H,)),
            full2((H, H)), full2((H,)),
            full2((H, H)), full2((H,)),
            full2((H, H)), full2((H,)),
        ],
        out_specs=pl.BlockSpec((1, BR, H), lambda b, k: (b, k, 0)),
        out_shape=jax.ShapeDtypeStruct((NBATCH, N_REAL, H), jnp.float32),
    )(aggp, h_prev, x, w_r, b_r, w_z, b_z, w_h, b_h, gcn_w, gcn_b)


def kernel(edge_index, x, h_prev, w_r, b_r, w_z, b_z, w_h, b_h, gcn_w, gcn_b):
    src = edge_index[0].astype(jnp.int32)
    dst = edge_index[1].astype(jnp.int32)
    pad_w = N_PADB - N_REAL
    src = src + pad_w * (src // N_REAL)
    dst = dst + pad_w * (dst // N_REAL)
    n_fill = ER * H - src.shape[0]
    fill = N_REAL + (jnp.arange(n_fill, dtype=jnp.int32) % pad_w)
    srcp = jnp.concatenate([src, fill]).reshape(ER, H)
    dstp = jnp.concatenate([dst, fill]).reshape(ER, H)

    hpad = jnp.pad(h_prev, ((0, 0), (0, pad_w), (0, 0))).reshape(NP, H)

    _, aggp = _sc_aggregate(srcp, dstp,
                            srcp.reshape(2 * ER, GR), dstp.reshape(2 * ER, GR),
                            hpad)
    return _tc_gru(aggp.reshape(NC, NP, H), h_prev, x,
                   w_r, b_r, w_z, b_z, w_h, b_h, gcn_w, gcn_b)


# speedup vs baseline: 1.0257x; 1.0257x over previous
import functools

import jax
import jax.numpy as jnp
from jax import lax
from jax.experimental import pallas as pl
from jax.experimental.pallas import tpu as pltpu
from jax.experimental.pallas import tpu_sc as plsc

NC, NS, L = 2, 16, 16
NBATCH = 4
N_REAL = 2500
N_PADB = 2560
NP = NBATCH * N_PADB
H = 128
ER = 2560
EPT = ER // NS
EPW = ER // (NS * NC)
RPT = NP // NS
CH = 16
EC = 16


def _fast_rsqrt(d):
    i = lax.bitcast_convert_type(d, jnp.int32)
    y = lax.bitcast_convert_type(jnp.int32(0x5F3759DF) - (i >> 1), jnp.float32)
    for _ in range(3):
        y = y * (1.5 - 0.5 * d * y * y)
    return y


def _sc_body(srcp, dstp, hpad, g_out, agg_out,
             sbuf, dbuf, hb0, hb1, rows_a, rows_b, ones_v,
             dv, ev, ns_v, nd_v, dout_sh, din_sh, acc_sh,
             sem_a, sem_b, sem_c, sem_d, sem_e):
    c = lax.axis_index("c")
    s = lax.axis_index("s")

    def _zrow(j, _):
        for k in range(8):
            hb0[j, pl.ds(k * L, L)] = jnp.zeros((L,), jnp.float32)
        return 0
    lax.fori_loop(0, CH, _zrow, 0)
    for k in range(8):
        ones_v[pl.ds(k * L, L)] = jnp.ones((L,), jnp.float32)

    zdescs = []
    for k in range(RPT // H):
        zdescs.append(pltpu.async_copy(
            hb0.at[0], dout_sh.at[pl.ds(s * RPT + k * H, H)], sem_c))
        zdescs.append(pltpu.async_copy(
            hb0.at[0], din_sh.at[pl.ds(s * RPT + k * H, H)], sem_c))
    for t in range(RPT // CH):
        zdescs.append(pltpu.async_copy(
            hb0, acc_sh.at[pl.ds(s * RPT + t * CH, CH)], sem_c))
    for d in zdescs:
        d.wait()
    plsc.subcore_barrier()

    for t in range(EPT // EC):
        pltpu.sync_copy(srcp.at[pl.ds(s * EPT + t * EC, EC)], sbuf)
        pltpu.sync_copy(dstp.at[pl.ds(s * EPT + t * EC, EC)], dbuf)
        descs = []
        for i in range(EC):
            descs.append(pltpu.async_copy(
                ones_v, dout_sh.at[sbuf.at[i]], sem_c, add=True))
            descs.append(pltpu.async_copy(
                ones_v, din_sh.at[dbuf.at[i]], sem_c, add=True))
        for d in descs:
            d.wait()
    plsc.subcore_barrier()

    pltpu.sync_copy(dout_sh.at[pl.ds(s * RPT, RPT)], dv)
    pltpu.sync_copy(din_sh.at[pl.ds(s * RPT, RPT)], ev)

    def _norm(i, _):
        sl = pl.ds(i * L, L)
        ns_v[sl] = _fast_rsqrt(jnp.maximum(dv[sl], 1.0))
        nd_v[sl] = _fast_rsqrt(jnp.maximum(ev[sl], 1.0))
        return 0
    lax.fori_loop(0, RPT // L, _norm, 0)

    hb = (hb0, hb1)
    isems = (sem_a, sem_b)
    osems = (sem_d, sem_e)
    NCH = RPT // CH
    din = [None, None]
    dout = [None, None]
    din[0] = pltpu.async_copy(hpad.at[pl.ds(s * RPT, CH)], hb0, sem_a)
    for t in range(NCH):
        cur = t % 2
        nxt = 1 - cur
        if t + 1 < NCH:
            if dout[nxt] is not None:
                dout[nxt].wait()
            din[nxt] = pltpu.async_copy(
                hpad.at[pl.ds(s * RPT + (t + 1) * CH, CH)], hb[nxt],
                isems[nxt])
        din[cur].wait()

        def _scale(j, _, t=t, cur=cur):
            idx = jnp.zeros((L,), jnp.int32) + (t * CH + j)
            sv = plsc.load_gather(ns_v, [idx])
            for k in range(8):
                sl = pl.ds(k * L, L)
                hb[cur][j, sl] = hb[cur][j, sl] * sv
            return 0
        lax.fori_loop(0, CH, _scale, 0)
        dout[cur] = pltpu.async_copy(
            hb[cur], g_out.at[pl.ds(s * RPT + t * CH, CH)], osems[cur])
    for d in dout:
        if d is not None:
            d.wait()
    plsc.subcore_barrier()

    bufs = (rows_a, rows_b)
    for t in range(EPW // EC):
        base = s * EPT + c * EPW + t * EC
        pltpu.sync_copy(srcp.at[pl.ds(base, EC)], sbuf)
        pltpu.sync_copy(dstp.at[pl.ds(base, EC)], dbuf)
        dg = [None, None]
        sc = [None, None]
        dg[0] = pltpu.async_copy(g_out.at[sbuf.at[0]], rows_a, sem_a)
        for i in range(EC):
            cur = i % 2
            nxt = 1 - cur
            if i + 1 < EC:
                if sc[nxt] is not None:
                    sc[nxt].wait()
                dg[nxt] = pltpu.async_copy(
                    g_out.at[sbuf.at[i + 1]], bufs[nxt], isems[nxt])
            dg[cur].wait()
            sc[cur] = pltpu.async_copy(
                bufs[cur], acc_sh.at[dbuf.at[i]], osems[cur], add=True)
        for d in sc:
            if d is not None:
                d.wait()
    plsc.subcore_barrier()

    din = [None, None]
    dout = [None, None]
    din[0] = pltpu.async_copy(acc_sh.at[pl.ds(s * RPT, CH)], hb0, sem_a)
    for t in range(NCH):
        cur = t % 2
        nxt = 1 - cur
        if t + 1 < NCH:
            if dout[nxt] is not None:
                dout[nxt].wait()
            din[nxt] = pltpu.async_copy(
                acc_sh.at[pl.ds(s * RPT + (t + 1) * CH, CH)], hb[nxt],
                isems[nxt])
        din[cur].wait()

        def _scale_out(j, _, t=t, cur=cur):
            idx = jnp.zeros((L,), jnp.int32) + (t * CH + j)
            sv = plsc.load_gather(nd_v, [idx])
            for k in range(8):
                sl = pl.ds(k * L, L)
                hb[cur][j, sl] = hb[cur][j, sl] * sv
            return 0
        lax.fori_loop(0, CH, _scale_out, 0)
        dout[cur] = pltpu.async_copy(
            hb[cur], agg_out.at[pl.ds(c * NP + s * RPT + t * CH, CH)],
            osems[cur])
    for d in dout:
        if d is not None:
            d.wait()


def _sc_aggregate(srcp, dstp, hpad):
    mesh = plsc.VectorSubcoreMesh(core_axis_name="c", subcore_axis_name="s")
    f = pl.kernel(
        _sc_body,
        out_type=(
            jax.ShapeDtypeStruct((NP, H), jnp.float32),
            jax.ShapeDtypeStruct((NC * NP, H), jnp.float32),
        ),
        mesh=mesh,
        compiler_params=pltpu.CompilerParams(needs_layout_passes=False),
        scratch_types=[
            pltpu.VMEM((EC, H), jnp.int32),
            pltpu.VMEM((EC, H), jnp.int32),
            pltpu.VMEM((CH, H), jnp.float32),
            pltpu.VMEM((CH, H), jnp.float32),
            pltpu.VMEM((H, H), jnp.float32),
            pltpu.VMEM((H, H), jnp.float32),
            pltpu.VMEM((H,), jnp.float32),
            pltpu.VMEM((RPT,), jnp.float32),
            pltpu.VMEM((RPT,), jnp.float32),
            pltpu.VMEM((RPT,), jnp.float32),
            pltpu.VMEM((RPT,), jnp.float32),
            pltpu.VMEM_SHARED((NP,), jnp.float32),
            pltpu.VMEM_SHARED((NP,), jnp.float32),
            pltpu.VMEM_SHARED((NP, H), jnp.float32),
            pltpu.SemaphoreType.DMA,
            pltpu.SemaphoreType.DMA,
            pltpu.SemaphoreType.DMA,
            pltpu.SemaphoreType.DMA,
            pltpu.SemaphoreType.DMA,
        ],
    )
    return f(srcp, dstp, hpad)


BR = 512


def _tc_body(agg_ref, h_ref, x_ref, wr_ref, br_ref, wz_ref, bz_ref,
             wh_ref, bh_ref, gw_ref, gb_ref, out_ref):
    a = agg_ref[0] + agg_ref[1]
    hc = jnp.dot(a, gw_ref[...], preferred_element_type=jnp.float32)
    hc = hc + gb_ref[...][None, :]

    x = x_ref[...]
    dn = (((1,), (1,)), ((), ()))
    xr = lax.dot_general(x, wr_ref[...], dn,
                         preferred_element_type=jnp.float32) + br_ref[...][None, :]
    xz = lax.dot_general(x, wz_ref[...], dn,
                         preferred_element_type=jnp.float32) + bz_ref[...][None, :]
    xh = lax.dot_general(x, wh_ref[...], dn,
                         preferred_element_type=jnp.float32) + bh_ref[...][None, :]

    bid = pl.program_id(0)
    onehot = (lax.broadcasted_iota(jnp.int32, (NBATCH, 1), 0) == bid
              ).astype(jnp.float32)
    xr_b = jnp.sum(xr * onehot, axis=0, keepdims=True)
    xz_b = jnp.sum(xz * onehot, axis=0, keepdims=True)
    xh_b = jnp.sum(xh * onehot, axis=0, keepdims=True)

    r_t = jax.nn.sigmoid(xr_b + hc)
    z_t = jax.nn.sigmoid(xz_b + hc)
    h_tilde = jnp.tanh(xh_b + r_t * hc)
    out_ref[0] = (1.0 - z_t) * h_ref[0] + z_t * h_tilde


def _tc_gru(aggp, h_prev, x, w_r, b_r, w_z, b_z, w_h, b_h, gcn_w, gcn_b):
    KB = N_PADB // BR
    full2 = lambda shape: pl.BlockSpec(
        shape, lambda b, k: tuple(0 for _ in shape))
    return pl.pallas_call(
        _tc_body,
        grid=(NBATCH, KB),
        in_specs=[
            pl.BlockSpec((NC, BR, H), lambda b, k: (0, b * KB + k, 0)),
            pl.BlockSpec((1, BR, H), lambda b, k: (b, k, 0)),
            full2((NBATCH, H)),
            full2((H, H)), full2((H,)),
            full2((H, H)), full2((H,)),
            full2((H, H)), full2((H,)),
            full2((H, H)), full2((H,)),
        ],
        out_specs=pl.BlockSpec((1, BR, H), lambda b, k: (b, k, 0)),
        out_shape=jax.ShapeDtypeStruct((NBATCH, N_REAL, H), jnp.float32),
    )(aggp, h_prev, x, w_r, b_r, w_z, b_z, w_h, b_h, gcn_w, gcn_b)


def kernel(edge_index, x, h_prev, w_r, b_r, w_z, b_z, w_h, b_h, gcn_w, gcn_b):
    src = edge_index[0].astype(jnp.int32)
    dst = edge_index[1].astype(jnp.int32)
    pad_w = N_PADB - N_REAL
    src = src + pad_w * (src // N_REAL)
    dst = dst + pad_w * (dst // N_REAL)
    n_fill = ER * H - src.shape[0]
    fill = N_REAL + (jnp.arange(n_fill, dtype=jnp.int32) % pad_w)
    srcp = jnp.concatenate([src, fill]).reshape(ER, H)
    dstp = jnp.concatenate([dst, fill]).reshape(ER, H)

    hpad = jnp.pad(h_prev, ((0, 0), (0, pad_w), (0, 0))).reshape(NP, H)

    _, aggp = _sc_aggregate(srcp, dstp, hpad)
    return _tc_gru(aggp.reshape(NC, NP, H), h_prev, x,
                   w_r, b_r, w_z, b_z, w_h, b_h, gcn_w, gcn_b)


# speedup vs baseline: 1.0355x; 1.0096x over previous
import functools

import jax
import jax.numpy as jnp
from jax import lax
from jax.experimental import pallas as pl
from jax.experimental.pallas import tpu as pltpu
from jax.experimental.pallas import tpu_sc as plsc

NC, NS, L = 2, 16, 16
NBATCH = 4
N_REAL = 2500
N_PADB = 2560
NP = NBATCH * N_PADB
H = 128
ER = 2560
EPT = ER // NS
EPW = ER // (NS * NC)
RPT = NP // NS
CH = 16
EC = 16


def _fast_rsqrt(d):
    i = lax.bitcast_convert_type(d, jnp.int32)
    y = lax.bitcast_convert_type(jnp.int32(0x5F3759DF) - (i >> 1), jnp.float32)
    for _ in range(3):
        y = y * (1.5 - 0.5 * d * y * y)
    return y


def _sc_body(srcp, dstp, hpad, g_out, agg_out, nd_out,
             sbuf, dbuf, hb0, hb1, rows_a, rows_b, ones_v,
             dv, ev, ns_v, nd_v, dout_sh, din_sh, acc_sh,
             sem_a, sem_b, sem_c, sem_d, sem_e):
    c = lax.axis_index("c")
    s = lax.axis_index("s")

    def _zrow(j, _):
        for k in range(8):
            hb0[j, pl.ds(k * L, L)] = jnp.zeros((L,), jnp.float32)
        return 0
    lax.fori_loop(0, CH, _zrow, 0)
    for k in range(8):
        ones_v[pl.ds(k * L, L)] = jnp.ones((L,), jnp.float32)

    zdescs = []
    for k in range(RPT // H):
        zdescs.append(pltpu.async_copy(
            hb0.at[0], dout_sh.at[pl.ds(s * RPT + k * H, H)], sem_c))
        zdescs.append(pltpu.async_copy(
            hb0.at[0], din_sh.at[pl.ds(s * RPT + k * H, H)], sem_c))
    for t in range(RPT // CH):
        zdescs.append(pltpu.async_copy(
            hb0, acc_sh.at[pl.ds(s * RPT + t * CH, CH)], sem_c))
    for d in zdescs:
        d.wait()
    plsc.subcore_barrier()

    for t in range(EPT // EC):
        pltpu.sync_copy(srcp.at[pl.ds(s * EPT + t * EC, EC)], sbuf)
        pltpu.sync_copy(dstp.at[pl.ds(s * EPT + t * EC, EC)], dbuf)
        descs = []
        for i in range(EC):
            descs.append(pltpu.async_copy(
                ones_v, dout_sh.at[sbuf.at[i]], sem_c, add=True))
            descs.append(pltpu.async_copy(
                ones_v, din_sh.at[dbuf.at[i]], sem_c, add=True))
        for d in descs:
            d.wait()
    plsc.subcore_barrier()

    pltpu.sync_copy(dout_sh.at[pl.ds(s * RPT, RPT)], dv)
    pltpu.sync_copy(din_sh.at[pl.ds(s * RPT, RPT)], ev)

    def _norm(i, _):
        sl = pl.ds(i * L, L)
        ns_v[sl] = _fast_rsqrt(jnp.maximum(dv[sl], 1.0))
        nd_v[sl] = _fast_rsqrt(jnp.maximum(ev[sl], 1.0))
        return 0
    lax.fori_loop(0, RPT // L, _norm, 0)
    pltpu.sync_copy(nd_v, nd_out.at[pl.ds(s * RPT, RPT)])

    hb = (hb0, hb1)
    isems = (sem_a, sem_b)
    osems = (sem_d, sem_e)
    NCH = RPT // CH
    din = [None, None]
    dout = [None, None]
    din[0] = pltpu.async_copy(hpad.at[pl.ds(s * RPT, CH)], hb0, sem_a)
    for t in range(NCH):
        cur = t % 2
        nxt = 1 - cur
        if t + 1 < NCH:
            if dout[nxt] is not None:
                dout[nxt].wait()
            din[nxt] = pltpu.async_copy(
                hpad.at[pl.ds(s * RPT + (t + 1) * CH, CH)], hb[nxt],
                isems[nxt])
        din[cur].wait()

        def _scale(j, _, t=t, cur=cur):
            idx = jnp.zeros((L,), jnp.int32) + (t * CH + j)
            sv = plsc.load_gather(ns_v, [idx])
            for k in range(8):
                sl = pl.ds(k * L, L)
                hb[cur][j, sl] = hb[cur][j, sl] * sv
            return 0
        lax.fori_loop(0, CH, _scale, 0)
        dout[cur] = pltpu.async_copy(
            hb[cur], g_out.at[pl.ds(s * RPT + t * CH, CH)], osems[cur])
    for d in dout:
        if d is not None:
            d.wait()
    plsc.subcore_barrier()

    bufs = (rows_a, rows_b)
    for t in range(EPW // EC):
        base = s * EPT + c * EPW + t * EC
        pltpu.sync_copy(srcp.at[pl.ds(base, EC)], sbuf)
        pltpu.sync_copy(dstp.at[pl.ds(base, EC)], dbuf)
        dg = [None, None]
        sc = [None, None]
        dg[0] = pltpu.async_copy(g_out.at[sbuf.at[0]], rows_a, sem_a)
        for i in range(EC):
            cur = i % 2
            nxt = 1 - cur
            if i + 1 < EC:
                if sc[nxt] is not None:
                    sc[nxt].wait()
                dg[nxt] = pltpu.async_copy(
                    g_out.at[sbuf.at[i + 1]], bufs[nxt], isems[nxt])
            dg[cur].wait()
            sc[cur] = pltpu.async_copy(
                bufs[cur], acc_sh.at[dbuf.at[i]], osems[cur], add=True)
        for d in sc:
            if d is not None:
                d.wait()
    plsc.subcore_barrier()

    pltpu.sync_copy(acc_sh.at[pl.ds(s * RPT, RPT)],
                    agg_out.at[pl.ds(c * NP + s * RPT, RPT)])


def _sc_aggregate(srcp, dstp, hpad):
    mesh = plsc.VectorSubcoreMesh(core_axis_name="c", subcore_axis_name="s")
    f = pl.kernel(
        _sc_body,
        out_type=(
            jax.ShapeDtypeStruct((NP, H), jnp.float32),
            jax.ShapeDtypeStruct((NC * NP, H), jnp.float32),
            jax.ShapeDtypeStruct((NP,), jnp.float32),
        ),
        mesh=mesh,
        compiler_params=pltpu.CompilerParams(needs_layout_passes=False),
        scratch_types=[
            pltpu.VMEM((EC, H), jnp.int32),
            pltpu.VMEM((EC, H), jnp.int32),
            pltpu.VMEM((CH, H), jnp.float32),
            pltpu.VMEM((CH, H), jnp.float32),
            pltpu.VMEM((H, H), jnp.float32),
            pltpu.VMEM((H, H), jnp.float32),
            pltpu.VMEM((H,), jnp.float32),
            pltpu.VMEM((RPT,), jnp.float32),
            pltpu.VMEM((RPT,), jnp.float32),
            pltpu.VMEM((RPT,), jnp.float32),
            pltpu.VMEM((RPT,), jnp.float32),
            pltpu.VMEM_SHARED((NP,), jnp.float32),
            pltpu.VMEM_SHARED((NP,), jnp.float32),
            pltpu.VMEM_SHARED((NP, H), jnp.float32),
            pltpu.SemaphoreType.DMA,
            pltpu.SemaphoreType.DMA,
            pltpu.SemaphoreType.DMA,
            pltpu.SemaphoreType.DMA,
            pltpu.SemaphoreType.DMA,
        ],
    )
    return f(srcp, dstp, hpad)


BR = 512


def _tc_body(agg_ref, nd_ref, h_ref, x_ref, wr_ref, br_ref, wz_ref, bz_ref,
             wh_ref, bh_ref, gw_ref, gb_ref, out_ref):
    a = agg_ref[0] + agg_ref[1]
    hc = jnp.dot(a, gw_ref[...], preferred_element_type=jnp.float32)
    hc = hc * nd_ref[...] + gb_ref[...][None, :]

    x = x_ref[...]
    dn = (((1,), (1,)), ((), ()))
    xr = lax.dot_general(x, wr_ref[...], dn,
                         preferred_element_type=jnp.float32) + br_ref[...][None, :]
    xz = lax.dot_general(x, wz_ref[...], dn,
                         preferred_element_type=jnp.float32) + bz_ref[...][None, :]
    xh = lax.dot_general(x, wh_ref[...], dn,
                         preferred_element_type=jnp.float32) + bh_ref[...][None, :]

    bid = pl.program_id(0)
    onehot = (lax.broadcasted_iota(jnp.int32, (NBATCH, 1), 0) == bid
              ).astype(jnp.float32)
    xr_b = jnp.sum(xr * onehot, axis=0, keepdims=True)
    xz_b = jnp.sum(xz * onehot, axis=0, keepdims=True)
    xh_b = jnp.sum(xh * onehot, axis=0, keepdims=True)

    r_t = jax.nn.sigmoid(xr_b + hc)
    z_t = jax.nn.sigmoid(xz_b + hc)
    h_tilde = jnp.tanh(xh_b + r_t * hc)
    out_ref[0] = (1.0 - z_t) * h_ref[0] + z_t * h_tilde


def _tc_gru(aggp, nd, h_prev, x, w_r, b_r, w_z, b_z, w_h, b_h, gcn_w, gcn_b):
    KB = N_PADB // BR
    full2 = lambda shape: pl.BlockSpec(
        shape, lambda b, k: tuple(0 for _ in shape))
    return pl.pallas_call(
        _tc_body,
        grid=(NBATCH, KB),
        in_specs=[
            pl.BlockSpec((NC, BR, H), lambda b, k: (0, b * KB + k, 0)),
            pl.BlockSpec((BR, 1), lambda b, k: (b * KB + k, 0)),
            pl.BlockSpec((1, BR, H), lambda b, k: (b, k, 0)),
            full2((NBATCH, H)),
            full2((H, H)), full2((H,)),
            full2((H, H)), full2((H,)),
            full2((H, H)), full2((H,)),
            full2((H, H)), full2((H,)),
        ],
        out_specs=pl.BlockSpec((1, BR, H), lambda b, k: (b, k, 0)),
        out_shape=jax.ShapeDtypeStruct((NBATCH, N_REAL, H), jnp.float32),
    )(aggp, nd, h_prev, x, w_r, b_r, w_z, b_z, w_h, b_h, gcn_w, gcn_b)


def kernel(edge_index, x, h_prev, w_r, b_r, w_z, b_z, w_h, b_h, gcn_w, gcn_b):
    src = edge_index[0].astype(jnp.int32)
    dst = edge_index[1].astype(jnp.int32)
    pad_w = N_PADB - N_REAL
    src = src + pad_w * (src // N_REAL)
    dst = dst + pad_w * (dst // N_REAL)
    n_fill = ER * H - src.shape[0]
    fill = N_REAL + (jnp.arange(n_fill, dtype=jnp.int32) % pad_w)
    srcp = jnp.concatenate([src, fill]).reshape(ER, H)
    dstp = jnp.concatenate([dst, fill]).reshape(ER, H)

    hpad = jnp.pad(h_prev, ((0, 0), (0, pad_w), (0, 0))).reshape(NP, H)
    _, aggp, nd = _sc_aggregate(srcp, dstp, hpad)
    return _tc_gru(aggp.reshape(NC, NP, H), nd.reshape(NP, 1), h_prev, x,
                   w_r, b_r, w_z, b_z, w_h, b_h, gcn_w, gcn_b)


# speedup vs baseline: 1.0367x; 1.0012x over previous
import jax
import jax.numpy as jnp
from jax import lax
from jax.experimental import pallas as pl
from jax.experimental.pallas import tpu as pltpu
from jax.experimental.pallas import tpu_sc as plsc

NC, NS, L = 2, 16, 16
NBATCH = 4
N_REAL = 2500
N_PADB = 2560
NP = NBATCH * N_PADB
H = 128
ER = 2560
EPT = ER // NS
EPW = ER // (NS * NC)
RPT = NP // NS
CH = 16
EC = 16


def _fast_rsqrt(d):
    i = lax.bitcast_convert_type(d, jnp.int32)
    y = lax.bitcast_convert_type(jnp.int32(0x5F3759DF) - (i >> 1), jnp.float32)
    for _ in range(3):
        y = y * (1.5 - 0.5 * d * y * y)
    return y


def _sc_body(srcp, dstp, hpad, g_out, agg_out, nd_out,
             sbuf, dbuf, hb0, hb1, rows_a, rows_b, ones_v,
             dv, ev, ns_v, nd_v, dout_sh, din_sh, acc_sh,
             sem_a, sem_b, sem_c, sem_d, sem_e):
    c = lax.axis_index("c")
    s = lax.axis_index("s")

    def _zrow(j, _):
        for k in range(8):
            hb0[j, pl.ds(k * L, L)] = jnp.zeros((L,), jnp.float32)
        return 0
    lax.fori_loop(0, CH, _zrow, 0)
    for k in range(8):
        ones_v[pl.ds(k * L, L)] = jnp.ones((L,), jnp.float32)

    zdescs = []
    for k in range(RPT // H):
        zdescs.append(pltpu.async_copy(
            hb0.at[0], dout_sh.at[pl.ds(s * RPT + k * H, H)], sem_c))
        zdescs.append(pltpu.async_copy(
            hb0.at[0], din_sh.at[pl.ds(s * RPT + k * H, H)], sem_c))
    for t in range(RPT // CH):
        zdescs.append(pltpu.async_copy(
            hb0, acc_sh.at[pl.ds(s * RPT + t * CH, CH)], sem_c))
    for d in zdescs:
        d.wait()
    plsc.subcore_barrier()

    for t in range(EPT // EC):
        pltpu.sync_copy(srcp.at[pl.ds(s * EPT + t * EC, EC)], sbuf)
        pltpu.sync_copy(dstp.at[pl.ds(s * EPT + t * EC, EC)], dbuf)
        descs = []
        for i in range(EC):
            descs.append(pltpu.async_copy(
                ones_v, dout_sh.at[sbuf.at[i]], sem_c, add=True))
            descs.append(pltpu.async_copy(
                ones_v, din_sh.at[dbuf.at[i]], sem_c, add=True))
        for d in descs:
            d.wait()
    plsc.subcore_barrier()

    pltpu.sync_copy(dout_sh.at[pl.ds(s * RPT, RPT)], dv)
    pltpu.sync_copy(din_sh.at[pl.ds(s * RPT, RPT)], ev)

    def _norm(i, _):
        sl = pl.ds(i * L, L)
        ns_v[sl] = _fast_rsqrt(jnp.maximum(dv[sl], 1.0))
        nd_v[sl] = _fast_rsqrt(jnp.maximum(ev[sl], 1.0))
        return 0
    lax.fori_loop(0, RPT // L, _norm, 0)
    pltpu.sync_copy(nd_v, nd_out.at[pl.ds(s * RPT, RPT)])

    hb = (hb0, hb1)
    isems = (sem_a, sem_b)
    osems = (sem_d, sem_e)
    NCH = RPT // CH
    din = [None, None]
    dout = [None, None]
    din[0] = pltpu.async_copy(hpad.at[pl.ds(s * RPT, CH)], hb0, sem_a)
    for t in range(NCH):
        cur = t % 2
        nxt = 1 - cur
        if t + 1 < NCH:
            if dout[nxt] is not None:
                dout[nxt].wait()
            din[nxt] = pltpu.async_copy(
                hpad.at[pl.ds(s * RPT + (t + 1) * CH, CH)], hb[nxt],
                isems[nxt])
        din[cur].wait()

        def _scale(j, _, t=t, cur=cur):
            idx = jnp.zeros((L,), jnp.int32) + (t * CH + j)
            sv = plsc.load_gather(ns_v, [idx])
            for k in range(8):
                sl = pl.ds(k * L, L)
                hb[cur][j, sl] = hb[cur][j, sl] * sv
            return 0
        lax.fori_loop(0, CH, _scale, 0)
        dout[cur] = pltpu.async_copy(
            hb[cur], g_out.at[pl.ds(s * RPT + t * CH, CH)], osems[cur])
    for d in dout:
        if d is not None:
            d.wait()
    plsc.subcore_barrier()

    bufs = (rows_a, rows_b)
    for t in range(EPW // EC):
        base = s * EPT + c * EPW + t * EC
        pltpu.sync_copy(srcp.at[pl.ds(base, EC)], sbuf)
        pltpu.sync_copy(dstp.at[pl.ds(base, EC)], dbuf)
        dg = [None, None]
        sc = [None, None]
        dg[0] = pltpu.async_copy(g_out.at[sbuf.at[0]], rows_a, sem_a)
        for i in range(EC):
            cur = i % 2
            nxt = 1 - cur
            if i + 1 < EC:
                if sc[nxt] is not None:
                    sc[nxt].wait()
                dg[nxt] = pltpu.async_copy(
                    g_out.at[sbuf.at[i + 1]], bufs[nxt], isems[nxt])
            dg[cur].wait()
            sc[cur] = pltpu.async_copy(
                bufs[cur], acc_sh.at[dbuf.at[i]], osems[cur], add=True)
        for d in sc:
            if d is not None:
                d.wait()
    plsc.subcore_barrier()

    pltpu.sync_copy(acc_sh.at[pl.ds(s * RPT, RPT)],
                    agg_out.at[pl.ds(c * NP + s * RPT, RPT)])


def _sc_aggregate(srcp, dstp, hpad):
    mesh = plsc.VectorSubcoreMesh(core_axis_name="c", subcore_axis_name="s")
    f = pl.kernel(
        _sc_body,
        out_type=(
            jax.ShapeDtypeStruct((NP, H), jnp.float32),
            jax.ShapeDtypeStruct((NC * NP, H), jnp.float32),
            jax.ShapeDtypeStruct((NP,), jnp.float32),
        ),
        mesh=mesh,
        compiler_params=pltpu.CompilerParams(needs_layout_passes=False),
        scratch_types=[
            pltpu.VMEM((EC, H), jnp.int32),
            pltpu.VMEM((EC, H), jnp.int32),
            pltpu.VMEM((CH, H), jnp.float32),
            pltpu.VMEM((CH, H), jnp.float32),
            pltpu.VMEM((H, H), jnp.float32),
            pltpu.VMEM((H, H), jnp.float32),
            pltpu.VMEM((H,), jnp.float32),
            pltpu.VMEM((RPT,), jnp.float32),
            pltpu.VMEM((RPT,), jnp.float32),
            pltpu.VMEM((RPT,), jnp.float32),
            pltpu.VMEM((RPT,), jnp.float32),
            pltpu.VMEM_SHARED((NP,), jnp.float32),
            pltpu.VMEM_SHARED((NP,), jnp.float32),
            pltpu.VMEM_SHARED((NP, H), jnp.float32),
            pltpu.SemaphoreType.DMA,
            pltpu.SemaphoreType.DMA,
            pltpu.SemaphoreType.DMA,
            pltpu.SemaphoreType.DMA,
            pltpu.SemaphoreType.DMA,
        ],
    )
    return f(srcp, dstp, hpad)


BR = 512


def _tc_body(agg_ref, nd_ref, h_ref, x_ref, wr_ref, br_ref, wz_ref, bz_ref,
             wh_ref, bh_ref, gw_ref, gb_ref, out_ref):
    a = agg_ref[0] + agg_ref[1]
    hc = jnp.dot(a, gw_ref[...], preferred_element_type=jnp.float32)
    hc = hc * nd_ref[...] + gb_ref[...][None, :]

    x = x_ref[...]
    dn = (((1,), (1,)), ((), ()))
    xr = lax.dot_general(x, wr_ref[...], dn,
                         preferred_element_type=jnp.float32) + br_ref[...][None, :]
    xz = lax.dot_general(x, wz_ref[...], dn,
                         preferred_element_type=jnp.float32) + bz_ref[...][None, :]
    xh = lax.dot_general(x, wh_ref[...], dn,
                         preferred_element_type=jnp.float32) + bh_ref[...][None, :]

    bid = pl.program_id(0)
    onehot = (lax.broadcasted_iota(jnp.int32, (NBATCH, 1), 0) == bid
              ).astype(jnp.float32)
    xr_b = jnp.sum(xr * onehot, axis=0, keepdims=True)
    xz_b = jnp.sum(xz * onehot, axis=0, keepdims=True)
    xh_b = jnp.sum(xh * onehot, axis=0, keepdims=True)

    r_t = jax.nn.sigmoid(xr_b + hc)
    z_t = jax.nn.sigmoid(xz_b + hc)
    h_tilde = jnp.tanh(xh_b + r_t * hc)
    out_ref[0] = (1.0 - z_t) * h_ref[0] + z_t * h_tilde


def _tc_gru(aggp, nd, h_prev, x, w_r, b_r, w_z, b_z, w_h, b_h, gcn_w, gcn_b):
    KB = N_PADB // BR
    full2 = lambda shape: pl.BlockSpec(
        shape, lambda b, k: tuple(0 for _ in shape))
    return pl.pallas_call(
        _tc_body,
        grid=(NBATCH, KB),
        in_specs=[
            pl.BlockSpec((NC, BR, H), lambda b, k: (0, b * KB + k, 0)),
            pl.BlockSpec((BR, 1), lambda b, k: (b * KB + k, 0)),
            pl.BlockSpec((1, BR, H), lambda b, k: (b, k, 0)),
            full2((NBATCH, H)),
            full2((H, H)), full2((H,)),
            full2((H, H)), full2((H,)),
            full2((H, H)), full2((H,)),
            full2((H, H)), full2((H,)),
        ],
        out_specs=pl.BlockSpec((1, BR, H), lambda b, k: (b, k, 0)),
        out_shape=jax.ShapeDtypeStruct((NBATCH, N_REAL, H), jnp.float32),
    )(aggp, nd, h_prev, x, w_r, b_r, w_z, b_z, w_h, b_h, gcn_w, gcn_b)


def kernel(edge_index, x, h_prev, w_r, b_r, w_z, b_z, w_h, b_h, gcn_w, gcn_b):
    src = edge_index[0].astype(jnp.int32)
    dst = edge_index[1].astype(jnp.int32)
    pad_w = N_PADB - N_REAL
    src = src + pad_w * (src // N_REAL)
    dst = dst + pad_w * (dst // N_REAL)
    n_fill = ER * H - src.shape[0]
    fill = N_REAL + (jnp.arange(n_fill, dtype=jnp.int32) % pad_w)
    srcp = jnp.concatenate([src, fill]).reshape(ER, H)
    dstp = jnp.concatenate([dst, fill]).reshape(ER, H)

    hpad = jnp.pad(h_prev, ((0, 0), (0, pad_w), (0, 0))).reshape(NP, H)
    _, aggp, nd = _sc_aggregate(srcp, dstp, hpad)
    return _tc_gru(aggp.reshape(NC, NP, H), nd.reshape(NP, 1), h_prev, x,
                   w_r, b_r, w_z, b_z, w_h, b_h, gcn_w, gcn_b)
